# baseline (speedup 1.0000x reference)
import functools

import jax
import jax.numpy as jnp
from jax import lax
from jax.experimental import pallas as pl
from jax.experimental.pallas import tpu as pltpu
from jax.experimental.pallas import tpu_sc as plsc


def _sc_scatter_kernel(NP, ED, E, NC, NS, CH):
    NW = NC * NS
    EPW = E // NW
    NCHUNK = EPW // CH
    RPT = NP // NS

    mesh = plsc.VectorSubcoreMesh(core_axis_name="c", subcore_axis_name="s")

    NBUF = 3

    @functools.partial(
        pl.kernel,
        out_type=jax.ShapeDtypeStruct((NC, NP, ED), jnp.float32),
        mesh=mesh,
        scratch_types=[
            pltpu.VMEM((NCHUNK, CH), jnp.int32),
            pltpu.VMEM((NBUF, CH, ED), jnp.float32),
            pltpu.VMEM_SHARED((NP, ED), jnp.float32),
            [pltpu.SemaphoreType.DMA] * NBUF,
        ],
    )
    def scatter(col_hbm, ea_hbm, zeros_hbm, out_hbm, idx_v, buf_v, agg_s, sems):
        c = lax.axis_index("c")
        s = lax.axis_index("s")
        wid = s * NC + c
        ebase = wid * EPW

        pltpu.sync_copy(zeros_hbm, agg_s.at[pl.ds(s * RPT, RPT)])
        pltpu.sync_copy(col_hbm.at[wid], idx_v)
        plsc.subcore_barrier()

        for b in range(NBUF):
            pltpu.async_copy(ea_hbm.at[pl.ds(ebase + b * CH, CH)],
                             buf_v.at[b], sems[b])

        def step(j, b):
            pltpu.make_async_copy(ea_hbm.at[pl.ds(ebase, CH)],
                                  buf_v.at[b], sems[b]).wait()
            pltpu.sync_copy(buf_v.at[b], agg_s.at[idx_v.at[j]], add=True)
            nj = j + NBUF

            @pl.when(nj < NCHUNK)
            def _():
                pltpu.async_copy(ea_hbm.at[pl.ds(ebase + nj * CH, CH)],
                                 buf_v.at[b], sems[b])

        def body(jq, carry):
            for b in range(NBUF):
                step(jq * NBUF + b, b)
            return carry

        NQ = NCHUNK // NBUF
        lax.fori_loop(0, NQ, body, 0)
        for b in range(NCHUNK % NBUF):
            step(jnp.int32(NQ * NBUF + b), b)
        plsc.subcore_barrier()

        pltpu.sync_copy(agg_s.at[pl.ds(s * RPT, RPT)],
                        out_hbm.at[c].at[pl.ds(s * RPT, RPT)])

    return scatter


def _ln(h, g, b):
    m = jnp.mean(h, axis=-1, keepdims=True)
    v = jnp.mean((h - m) ** 2, axis=-1, keepdims=True)
    return (h - m) * lax.rsqrt(v + 1e-5) * g + b


def _gelu(h):
    return h * 0.5 * (1.0 + lax.erf(h * (2.0 ** -0.5)))


def _mlp_a_body(x_ref, oh_ref, u_ref, w1a, w1c, b1, h1p_ref):
    uw = jnp.dot(u_ref[...], w1c[...], preferred_element_type=jnp.float32)
    h1p_ref[...] = (jnp.dot(x_ref[...], w1a[...],
                            preferred_element_type=jnp.float32)
                    + jnp.dot(oh_ref[...], uw,
                              preferred_element_type=jnp.float32)
                    + b1[...]).astype(jnp.bfloat16)


def _lngelu(h, g, b):
    m = jnp.mean(h, axis=-1, keepdims=True)
    v = jnp.mean((h - m) ** 2, axis=-1, keepdims=True)
    a = (h - m) * lax.rsqrt(v + 1e-5) * g + b
    u = a * (2.0 ** -0.5)
    return u + u * lax.erf(a)


def _mlp_body(h1p_ref, aggp_ref,
              w1b, g1, be1,
              w2, b2, g2, be2,
              w3, b3, g3, be3,
              w4, b4, out_ref):
    agg = aggp_ref[0] + aggp_ref[1]
    h = (h1p_ref[...].astype(jnp.float32)
         + jnp.dot(agg, w1b[...], preferred_element_type=jnp.float32))
    h = _lngelu(h, g1[...], be1[...])
    h = jnp.dot(h, w2[...], preferred_element_type=jnp.float32) + b2[...]
    h = _lngelu(h, g2[...], be2[...])
    h = jnp.dot(h, w3[...], preferred_element_type=jnp.float32) + b3[...]
    h = _lngelu(h, g3[...], be3[...])
    out_ref[...] = jnp.dot(h, w4[...], preferred_element_type=jnp.float32) + b4[...]


def kernel(x, edge_index, edge_attr, u, batch,
           W1, b1, g1, be1, W2, b2, g2, be2, W3, b3, g3, be3, W4, b4):
    N, ND = x.shape
    E, ED = edge_attr.shape
    B, GD = u.shape
    H = W2.shape[0]
    OUT = W4.shape[1]

    NC, NS, CH = 2, 16, 80
    NW = NC * NS
    EPW = E // NW
    NCHUNK = EPW // CH

    NP = ((N + 8 * NS - 1) // (8 * NS)) * (8 * NS)
    col = edge_index[1].reshape(NW, NCHUNK, CH)
    zeros = jnp.zeros((NP // NS, ED), jnp.float32)
    aggp = _sc_scatter_kernel(NP, ED, E, NC, NS, CH)(col, edge_attr, zeros)

    oh = (batch[:, None] == jnp.arange(B, dtype=batch.dtype)[None, :]
          ).astype(jnp.float32)

    isq = 2.0 ** -0.5
    R = 2000
    NBLK = N // R
    full = lambda shape: pl.BlockSpec(shape, lambda i: (0,) * len(shape))
    row2 = lambda d: pl.BlockSpec((R, d), lambda i: (i, 0))

    h1p = pl.pallas_call(
        _mlp_a_body,
        grid=(NBLK,),
        in_specs=[
            row2(ND),
            row2(B),
            full((B, GD)),
            full((ND, H)), full((GD, H)),
            full((1, H)),
        ],
        out_specs=row2(H),
        out_shape=jax.ShapeDtypeStruct((N, H), jnp.bfloat16),
    )(x, oh, u, W1[:ND], W1[ND + ED:], b1.reshape(1, H))

    out = pl.pallas_call(
        _mlp_body,
        grid=(NBLK,),
        in_specs=[
            row2(H),
            pl.BlockSpec((NC, R, ED), lambda i: (0, i, 0)),
            full((ED, H)),
            full((1, H)), full((1, H)),
            full((H, H)), full((1, H)), full((1, H)), full((1, H)),
            full((H, H)), full((1, H)), full((1, H)), full((1, H)),
            full((H, OUT)), full((1, OUT)),
        ],
        out_specs=pl.BlockSpec((R, OUT), lambda i: (i, 0)),
        out_shape=jax.ShapeDtypeStruct((N, OUT), jnp.float32),
    )(h1p, aggp,
      W1[ND:ND + ED],
      (g1 * isq).reshape(1, H), (be1 * isq).reshape(1, H),
      W2, b2.reshape(1, H), (g2 * isq).reshape(1, H), (be2 * isq).reshape(1, H),
      W3, b3.reshape(1, H), (g3 * isq).reshape(1, H), (be3 * isq).reshape(1, H),
      W4, b4.reshape(1, OUT))
    return out


# speedup vs baseline: 1.0503x; 1.0503x over previous
import functools

import jax
import jax.numpy as jnp
from jax import lax
from jax.experimental import pallas as pl
from jax.experimental.pallas import tpu as pltpu
from jax.experimental.pallas import tpu_sc as plsc


def _sc_scatter_kernel(NP, ED, E, NC, NS, CH):
    NW = NC * NS
    EPW = E // NW
    NCHUNK = EPW // CH
    RPT = NP // NS

    mesh = plsc.VectorSubcoreMesh(core_axis_name="c", subcore_axis_name="s")

    NBUF = 3

    @functools.partial(
        pl.kernel,
        out_type=jax.ShapeDtypeStruct((NC, NP, ED), jnp.float32),
        mesh=mesh,
        scratch_types=[
            pltpu.VMEM((NCHUNK, CH), jnp.int32),
            pltpu.VMEM((NBUF, CH, ED), jnp.float32),
            pltpu.VMEM_SHARED((NP, ED), jnp.float32),
            [pltpu.SemaphoreType.DMA] * NBUF,
        ],
    )
    def scatter(col_hbm, ea_hbm, out_hbm, idx_v, buf_v, agg_s, sems):
        c = lax.axis_index("c")
        s = lax.axis_index("s")
        wid = s * NC + c
        ebase = wid * EPW

        z16 = jnp.zeros((16,), jnp.float32)

        def zrow(r, carry):
            for cseg in range(ED // 16):
                buf_v[0, r, pl.ds(cseg * 16, 16)] = z16
            return carry

        lax.fori_loop(0, CH, zrow, 0)
        for k in range(RPT // CH):
            pltpu.sync_copy(buf_v.at[0], agg_s.at[pl.ds(s * RPT + k * CH, CH)])
        pltpu.sync_copy(col_hbm.at[wid], idx_v)
        plsc.subcore_barrier()

        for b in range(NBUF):
            pltpu.async_copy(ea_hbm.at[pl.ds(ebase + b * CH, CH)],
                             buf_v.at[b], sems[b])

        def step(j, b):
            pltpu.make_async_copy(ea_hbm.at[pl.ds(ebase, CH)],
                                  buf_v.at[b], sems[b]).wait()
            pltpu.sync_copy(buf_v.at[b], agg_s.at[idx_v.at[j]], add=True)
            nj = j + NBUF

            @pl.when(nj < NCHUNK)
            def _():
                pltpu.async_copy(ea_hbm.at[pl.ds(ebase + nj * CH, CH)],
                                 buf_v.at[b], sems[b])

        def body(jq, carry):
            for b in range(NBUF):
                step(jq * NBUF + b, b)
            return carry

        NQ = NCHUNK // NBUF
        lax.fori_loop(0, NQ, body, 0)
        for b in range(NCHUNK % NBUF):
            step(jnp.int32(NQ * NBUF + b), b)
        plsc.subcore_barrier()

        pltpu.sync_copy(agg_s.at[pl.ds(s * RPT, RPT)],
                        out_hbm.at[c].at[pl.ds(s * RPT, RPT)])

    return scatter


def _ln(h, g, b):
    m = jnp.mean(h, axis=-1, keepdims=True)
    v = jnp.mean((h - m) ** 2, axis=-1, keepdims=True)
    return (h - m) * lax.rsqrt(v + 1e-5) * g + b


def _gelu(h):
    return h * 0.5 * (1.0 + lax.erf(h * (2.0 ** -0.5)))


def _mlp_a_body(x_ref, oh_ref, u_ref, w1a, w1c, b1, h1p_ref):
    uw = jnp.dot(u_ref[...], w1c[...], preferred_element_type=jnp.float32)
    h1p_ref[...] = (jnp.dot(x_ref[...], w1a[...],
                            preferred_element_type=jnp.float32)
                    + jnp.dot(oh_ref[...], uw,
                              preferred_element_type=jnp.float32)
                    + b1[...]).astype(jnp.bfloat16)


def _lngelu(h, g, b):
    m = jnp.mean(h, axis=-1, keepdims=True)
    v = jnp.mean((h - m) ** 2, axis=-1, keepdims=True)
    a = (h - m) * lax.rsqrt(v + 1e-5) * g + b
    u = a * (2.0 ** -0.5)
    return u + u * lax.erf(a)


def _mlp_body(h1p_ref, aggp_ref,
              w1b, g1, be1,
              w2, b2, g2, be2,
              w3, b3, g3, be3,
              w4, b4, out_ref):
    agg = aggp_ref[0] + aggp_ref[1]
    h = (h1p_ref[...].astype(jnp.float32)
         + jnp.dot(agg, w1b[...], preferred_element_type=jnp.float32))
    h = _lngelu(h, g1[...], be1[...])
    h = jnp.dot(h, w2[...], preferred_element_type=jnp.float32) + b2[...]
    h = _lngelu(h, g2[...], be2[...])
    h = jnp.dot(h, w3[...], preferred_element_type=jnp.float32) + b3[...]
    h = _lngelu(h, g3[...], be3[...])
    out_ref[...] = jnp.dot(h, w4[...], preferred_element_type=jnp.float32) + b4[...]


def kernel(x, edge_index, edge_attr, u, batch,
           W1, b1, g1, be1, W2, b2, g2, be2, W3, b3, g3, be3, W4, b4):
    N, ND = x.shape
    E, ED = edge_attr.shape
    B, GD = u.shape
    H = W2.shape[0]
    OUT = W4.shape[1]

    NC, NS, CH = 2, 16, 80
    NW = NC * NS
    EPW = E // NW
    NCHUNK = EPW // CH

    NP = ((N + 8 * NS - 1) // (8 * NS)) * (8 * NS)
    col = edge_index[1].reshape(NW, NCHUNK, CH)
    aggp = _sc_scatter_kernel(NP, ED, E, NC, NS, CH)(col, edge_attr)

    oh = (batch[:, None] == jnp.arange(B, dtype=batch.dtype)[None, :]
          ).astype(jnp.float32)

    isq = 2.0 ** -0.5
    R = 1000
    NBLK = N // R
    full = lambda shape: pl.BlockSpec(shape, lambda i: (0,) * len(shape))
    row2 = lambda d: pl.BlockSpec((R, d), lambda i: (i, 0))

    h1p = pl.pallas_call(
        _mlp_a_body,
        grid=(NBLK,),
        in_specs=[
            row2(ND),
            row2(B),
            full((B, GD)),
            full((ND, H)), full((GD, H)),
            full((1, H)),
        ],
        out_specs=row2(H),
        out_shape=jax.ShapeDtypeStruct((N, H), jnp.bfloat16),
    )(x, oh, u, W1[:ND], W1[ND + ED:], b1.reshape(1, H))

    out = pl.pallas_call(
        _mlp_body,
        grid=(NBLK,),
        in_specs=[
            row2(H),
            pl.BlockSpec((NC, R, ED), lambda i: (0, i, 0)),
            full((ED, H)),
            full((1, H)), full((1, H)),
            full((H, H)), full((1, H)), full((1, H)), full((1, H)),
            full((H, H)), full((1, H)), full((1, H)), full((1, H)),
            full((H, OUT)), full((1, OUT)),
        ],
        out_specs=pl.BlockSpec((R, OUT), lambda i: (i, 0)),
        out_shape=jax.ShapeDtypeStruct((N, OUT), jnp.float32),
    )(h1p, aggp,
      W1[ND:ND + ED],
      (g1 * isq).reshape(1, H), (be1 * isq).reshape(1, H),
      W2, b2.reshape(1, H), (g2 * isq).reshape(1, H), (be2 * isq).reshape(1, H),
      W3, b3.reshape(1, H), (g3 * isq).reshape(1, H), (be3 * isq).reshape(1, H),
      W4, b4.reshape(1, OUT))
    return out
